# baseline (speedup 1.0000x reference)
import jax
import jax.numpy as jnp
from jax.experimental import pallas as pl
from jax.experimental.pallas import tpu as pltpu

N = 16384
C = 1000
K = int(0.7 * N)
IGNORE = 0

B = 2048
NB = N // B
CH = 8
NCH = C // CH


def _body(xt_ref, tgt_ref, out_ref, loss_ref):
    i = pl.program_id(0)
    tgt = tgt_ref[...]

    def mbody(j, acc):
        return jnp.maximum(acc, xt_ref[pl.ds(j * CH, CH), :])

    accm = jax.lax.fori_loop(
        0, NCH, mbody, jnp.full((CH, B), -jnp.inf, jnp.float32))
    m = jnp.max(accm, axis=0, keepdims=True)

    rows8 = jax.lax.broadcasted_iota(jnp.int32, (CH, B), 0)

    def ebody(j, carry):
        s_acc, p_acc = carry
        chunk = xt_ref[pl.ds(j * CH, CH), :]
        pick = jnp.where(rows8 + j * CH == tgt, chunk, 0.0)
        return s_acc + jnp.exp(chunk - m), p_acc + pick

    s8, p8 = jax.lax.fori_loop(
        0, NCH, ebody,
        (jnp.zeros((CH, B), jnp.float32), jnp.zeros((CH, B), jnp.float32)))
    s = jnp.sum(s8, axis=0, keepdims=True)
    picked = jnp.sum(p8, axis=0, keepdims=True)
    lse = m + jnp.log(s)
    loss_ref[pl.ds(i, 1), :] = jnp.where(tgt == IGNORE, 0.0, lse - picked)

    @pl.when(i == NB - 1)
    def _select():
        lx = loss_ref[...]
        bits = jax.lax.bitcast_convert_type(lx, jnp.int32)

        def srch(_, carry):
            lo, hi = carry
            mid = lo + (hi - lo) // 2
            cnt = jnp.sum(jnp.where(bits >= mid, 1, 0))
            return (jnp.where(cnt >= K, mid, lo),
                    jnp.where(cnt >= K, hi, mid))

        t, _ = jax.lax.fori_loop(
            0, 31, srch, (jnp.int32(0), jnp.int32(0x7F800001)))
        gt = bits > t
        cnt_gt = jnp.sum(jnp.where(gt, 1.0, 0.0))
        sum_gt = jnp.sum(jnp.where(gt, lx, 0.0))
        tv = jnp.max(jax.lax.bitcast_convert_type(
            jnp.full((8, 128), t, jnp.int32), jnp.float32))
        out_ref[0, 0] = (sum_gt + (jnp.float32(K) - cnt_gt) * tv) * (1.0 / K)


@jax.jit
def kernel(input, target):
    xt = input.T
    tgt2d = target.astype(jnp.int32)[None, :]

    out = pl.pallas_call(
        _body,
        grid=(NB,),
        in_specs=[
            pl.BlockSpec((C, B), lambda i: (0, i)),
            pl.BlockSpec((1, B), lambda i: (0, i)),
        ],
        out_specs=pl.BlockSpec(memory_space=pltpu.SMEM),
        out_shape=jax.ShapeDtypeStruct((1, 1), jnp.float32),
        scratch_shapes=[pltpu.VMEM((NB, B), jnp.float32)],
    )(xt, tgt2d)
    return out[0, 0]


# speedup vs baseline: 1.5603x; 1.5603x over previous
import jax
import jax.numpy as jnp
from jax.experimental import pallas as pl
from jax.experimental.pallas import tpu as pltpu

N = 16384
C = 1000
K = int(0.7 * N)
IGNORE = 0

B = 2048
NB = N // B


def _body(xt_ref, tgt_ref, out_ref, loss_ref):
    i = pl.program_id(0)
    x = xt_ref[...]
    tgt = tgt_ref[...]
    m = jnp.max(x, axis=0, keepdims=True)
    s = jnp.sum(jnp.exp(x - m), axis=0, keepdims=True)
    lse = m + jnp.log(s)
    rows = jax.lax.broadcasted_iota(jnp.int32, (C, B), 0)
    picked = jnp.sum(jnp.where(rows == tgt, x, 0.0), axis=0, keepdims=True)
    loss_ref[pl.ds(i, 1), :] = jnp.where(tgt == IGNORE, 0.0, lse - picked)

    @pl.when(i == NB - 1)
    def _select():
        lx = loss_ref[...]
        bits = jax.lax.bitcast_convert_type(lx, jnp.int32)

        def srch(_, carry):
            lo, hi = carry
            mid = lo + (hi - lo) // 2
            cnt = jnp.sum(jnp.where(bits >= mid, 1, 0))
            return (jnp.where(cnt >= K, mid, lo),
                    jnp.where(cnt >= K, hi, mid))

        t, _ = jax.lax.fori_loop(
            0, 31, srch, (jnp.int32(0), jnp.int32(0x7F800001)))
        gt = bits > t
        cnt_gt = jnp.sum(jnp.where(gt, 1.0, 0.0))
        sum_gt = jnp.sum(jnp.where(gt, lx, 0.0))
        tv = jnp.max(jax.lax.bitcast_convert_type(
            jnp.full((8, 128), t, jnp.int32), jnp.float32))
        out_ref[0, 0] = (sum_gt + (jnp.float32(K) - cnt_gt) * tv) * (1.0 / K)


@jax.jit
def kernel(input, target):
    xt = input.T
    tgt2d = target.astype(jnp.int32)[None, :]

    out = pl.pallas_call(
        _body,
        grid=(NB,),
        in_specs=[
            pl.BlockSpec((C, B), lambda i: (0, i)),
            pl.BlockSpec((1, B), lambda i: (0, i)),
        ],
        out_specs=pl.BlockSpec(memory_space=pltpu.SMEM),
        out_shape=jax.ShapeDtypeStruct((1, 1), jnp.float32),
        scratch_shapes=[pltpu.VMEM((NB, B), jnp.float32)],
    )(xt, tgt2d)
    return out[0, 0]
